# Initial kernel scaffold; baseline (speedup 1.0000x reference)
import jax
import jax.numpy as jnp
from jax.experimental import pallas as pl


def kernel(x, weights):
    raise NotImplementedError("write your pallas kernel here")


# speedup vs baseline: 28.6555x; 28.6555x over previous
import jax
import jax.numpy as jnp
from jax.experimental import pallas as pl


def _som_dist_kernel(x_ref, w_ref, o_ref):
    x = x_ref[...]
    w = w_ref[...]
    xw = jax.lax.dot_general(
        x, w,
        dimension_numbers=(((1,), (1,)), ((), ())),
        preferred_element_type=jnp.float32,
    )
    x2 = jnp.sum(x * x, axis=1, keepdims=True)
    w2 = jnp.sum(w * w, axis=1, keepdims=True).T
    o_ref[...] = x2 + w2 - 2.0 * xw


def kernel(x, weights):
    B, D = x.shape
    R, C, _ = weights.shape
    w = weights.reshape(R * C, D)
    out = pl.pallas_call(
        _som_dist_kernel,
        out_shape=jax.ShapeDtypeStruct((B, R * C), jnp.float32),
    )(x, w)
    return out.reshape(B, R, C)
